# Initial kernel scaffold; baseline (speedup 1.0000x reference)
import jax
import jax.numpy as jnp
from jax.experimental import pallas as pl


def kernel(input, weight):
    raise NotImplementedError("write your pallas kernel here")


# speedup vs baseline: 1.0211x; 1.0211x over previous
import functools

import jax
import jax.numpy as jnp
from jax import lax
from jax.experimental import pallas as pl
from jax.experimental.pallas import tpu as pltpu
from jax.experimental.pallas import tpu_sc as plsc

_NUM_TILES = 32
_GROUP = 128
_B = 16384 * 50
_D = 32
_GROUPS_PER_TILE = _B // (_NUM_TILES * _GROUP)

_mesh = plsc.VectorSubcoreMesh(core_axis_name="c", subcore_axis_name="s")


@functools.partial(
    pl.kernel,
    mesh=_mesh,
    compiler_params=pltpu.CompilerParams(use_tc_tiling_on_sc=False),
    out_type=jax.ShapeDtypeStruct((_B, _D), jnp.float32),
    scratch_types=[
        pltpu.VMEM((_GROUPS_PER_TILE, _GROUP), jnp.int32),
        pltpu.VMEM((_GROUP, _D), jnp.float32),
        pltpu.SemaphoreType.DMA,
    ],
)
def _gather_kernel(idx_hbm, table_hbm, out_hbm, idx_v, rows_v, sem):
    wid = lax.axis_index("s") * 2 + lax.axis_index("c")
    gbase = wid * _GROUPS_PER_TILE
    pltpu.sync_copy(idx_hbm.at[pl.ds(gbase, _GROUPS_PER_TILE)], idx_v)

    def body(g, carry):
        pltpu.async_copy(table_hbm.at[idx_v.at[g]], rows_v, sem).wait()
        pltpu.sync_copy(rows_v, out_hbm.at[pl.ds((gbase + g) * _GROUP, _GROUP)])
        return carry

    lax.fori_loop(0, _GROUPS_PER_TILE, body, 0)


def kernel(input, weight):
    idx = input.reshape(-1).astype(jnp.int32).reshape(_B // _GROUP, _GROUP)
    out = _gather_kernel(idx, weight)
    return out.reshape(tuple(input.shape) + tuple(weight.shape[1:]))


# speedup vs baseline: 1.1105x; 1.0875x over previous
import functools

import jax
import jax.numpy as jnp
from jax import lax
from jax.experimental import pallas as pl
from jax.experimental.pallas import tpu as pltpu
from jax.experimental.pallas import tpu_sc as plsc

_NUM_TILES = 32
_GROUP = 128
_B = 16384 * 50
_D = 32
_GROUPS_PER_TILE = _B // (_NUM_TILES * _GROUP)
_NBUF = 8
_NSTEPS = _GROUPS_PER_TILE // _NBUF

_mesh = plsc.VectorSubcoreMesh(core_axis_name="c", subcore_axis_name="s")


@functools.partial(
    pl.kernel,
    mesh=_mesh,
    compiler_params=pltpu.CompilerParams(use_tc_tiling_on_sc=False),
    out_type=jax.ShapeDtypeStruct((_B, _D), jnp.float32),
    scratch_types=(
        [pltpu.VMEM((_GROUPS_PER_TILE, _GROUP), jnp.int32),
         pltpu.VMEM((_NBUF, _GROUP, _D), jnp.float32)]
        + [pltpu.SemaphoreType.DMA] * (2 * _NBUF)
    ),
)
def _gather_kernel(idx_hbm, table_hbm, out_hbm, idx_v, rows_v, *sems):
    gsems = sems[:_NBUF]
    osems = sems[_NBUF:]
    wid = lax.axis_index("s") * 2 + lax.axis_index("c")
    gbase = wid * _GROUPS_PER_TILE
    pltpu.sync_copy(idx_hbm.at[pl.ds(gbase, _GROUPS_PER_TILE)], idx_v)

    def _gather(g, b):
        return pltpu.make_async_copy(
            table_hbm.at[idx_v.at[g]], rows_v.at[b], gsems[b])

    def _out(g, b):
        return pltpu.make_async_copy(
            rows_v.at[b], out_hbm.at[pl.ds((gbase + g) * _GROUP, _GROUP)],
            osems[b])

    for b in range(_NBUF):
        _gather(b, b).start()

    @pl.loop(0, _NSTEPS - 1)
    def _steps(s):
        for b in range(_NBUF):
            g = s * _NBUF + b
            _gather(g, b).wait()
            _out(g, b).start()
        for b in range(_NBUF):
            g = s * _NBUF + b
            _out(g, b).wait()
            _gather(g + _NBUF, b).start()

    last = (_NSTEPS - 1) * _NBUF
    for b in range(_NBUF):
        _gather(last + b, b).wait()
        _out(last + b, b).start()
    for b in range(_NBUF):
        _out(last + b, b).wait()


def kernel(input, weight):
    idx = input.reshape(-1).astype(jnp.int32).reshape(_B // _GROUP, _GROUP)
    out = _gather_kernel(idx, weight)
    return out.reshape(tuple(input.shape) + tuple(weight.shape[1:]))


# speedup vs baseline: 1.1107x; 1.0002x over previous
import functools

import jax
import jax.numpy as jnp
from jax import lax
from jax.experimental import pallas as pl
from jax.experimental.pallas import tpu as pltpu
from jax.experimental.pallas import tpu_sc as plsc

_NUM_TILES = 32
_GROUP = 128
_B = 16384 * 50
_D = 32
_GROUPS_PER_TILE = _B // (_NUM_TILES * _GROUP)
_NSLOT = 4
_NSTEP = _GROUPS_PER_TILE // _NSLOT

_mesh = plsc.VectorSubcoreMesh(core_axis_name="c", subcore_axis_name="s")


@functools.partial(
    pl.kernel,
    mesh=_mesh,
    compiler_params=pltpu.CompilerParams(use_tc_tiling_on_sc=False),
    out_type=jax.ShapeDtypeStruct((_B, _D), jnp.float32),
    scratch_types=(
        [pltpu.VMEM((_GROUPS_PER_TILE, _GROUP), jnp.int32),
         pltpu.VMEM((2 * _NSLOT, _GROUP, _D), jnp.float32)]
        + [pltpu.SemaphoreType.DMA] * (4 * _NSLOT)
    ),
)
def _gather_kernel(idx_hbm, table_hbm, out_hbm, idx_v, rows_v, *sems):
    gsems = sems[:2 * _NSLOT]
    osems = sems[2 * _NSLOT:]
    wid = lax.axis_index("s") * 2 + lax.axis_index("c")
    gbase = wid * _GROUPS_PER_TILE
    pltpu.sync_copy(idx_hbm.at[pl.ds(gbase, _GROUPS_PER_TILE)], idx_v)

    def _gather(g, b):
        return pltpu.make_async_copy(
            table_hbm.at[idx_v.at[g]], rows_v.at[b], gsems[b])

    def _out(g, b):
        return pltpu.make_async_copy(
            rows_v.at[b], out_hbm.at[pl.ds((gbase + g) * _GROUP, _GROUP)],
            osems[b])

    def _start_gathers(s, h):
        for j in range(_NSLOT):
            _gather(s * _NSLOT + j, h * _NSLOT + j).start()

    def _drain_gathers_start_outs(s, h):
        for j in range(_NSLOT):
            b = h * _NSLOT + j
            _gather(s * _NSLOT + j, b).wait()
            _out(s * _NSLOT + j, b).start()

    def _wait_outs(s, h):
        for j in range(_NSLOT):
            _out(s * _NSLOT + j, h * _NSLOT + j).wait()

    _start_gathers(0, 0)
    _start_gathers(1, 1)
    _drain_gathers_start_outs(0, 0)

    @pl.loop(0, (_NSTEP - 2) // 2)
    def _steps(it):
        sa = 2 + 2 * it
        _wait_outs(sa - 2, 0)
        _start_gathers(sa, 0)
        _drain_gathers_start_outs(sa - 1, 1)
        sb = sa + 1
        _wait_outs(sb - 2, 1)
        _start_gathers(sb, 1)
        _drain_gathers_start_outs(sb - 1, 0)

    _drain_gathers_start_outs(_NSTEP - 1, 1)
    _wait_outs(_NSTEP - 2, 0)
    _wait_outs(_NSTEP - 1, 1)


def kernel(input, weight):
    idx = input.reshape(-1).astype(jnp.int32).reshape(_B // _GROUP, _GROUP)
    out = _gather_kernel(idx, weight)
    return out.reshape(tuple(input.shape) + tuple(weight.shape[1:]))
